# baseline (speedup 1.0000x reference)
import functools

import jax
import jax.numpy as jnp
from jax import lax
from jax.experimental import pallas as pl
from jax.experimental.pallas import tpu as pltpu
from jax.experimental.pallas import tpu_sc as plsc

_BM = 2048


def _vq_body(z_ref, cb_ref, q_ref):
    zb = z_ref[...]
    cb = cb_ref[...]
    zn = zb / jnp.maximum(
        jnp.sqrt(jnp.sum(zb * zb, axis=-1, keepdims=True)), 1e-12)
    en = cb / jnp.maximum(
        jnp.sqrt(jnp.sum(cb * cb, axis=-1, keepdims=True)), 1e-12)
    sims = jax.lax.dot_general(
        zn, en, (((1,), (1,)), ((), ())),
        precision=jax.lax.Precision.DEFAULT)
    k = sims.shape[1]
    col = jax.lax.broadcasted_iota(jnp.int32, sims.shape, 1)
    sims = jnp.where(col == 0, -jnp.inf, sims)
    m = jnp.max(sims, axis=1, keepdims=True)
    qi = jnp.min(jnp.where(sims == m, col, k), axis=1)
    q_ref[...] = qi


def _argmax_tc(zf, codebook):
    m, d = zf.shape
    k = codebook.shape[0]
    return pl.pallas_call(
        _vq_body,
        grid=(m // _BM,),
        in_specs=[
            pl.BlockSpec((_BM, d), lambda i: (i, 0)),
            pl.BlockSpec((k, d), lambda i: (0, 0)),
        ],
        out_specs=pl.BlockSpec((_BM,), lambda i: (i,)),
        out_shape=jax.ShapeDtypeStruct((m,), jnp.int32),
    )(zf, codebook)


def _gather_sc(codebook, idx_flat):
    m = idx_flat.shape[0]
    d = codebook.shape[1]
    nw = 2 * 16
    bpw = m // nw
    chunk = 96
    nch = bpw // chunk
    mesh = plsc.VectorSubcoreMesh(core_axis_name="c", subcore_axis_name="s")

    @functools.partial(
        pl.kernel, mesh=mesh,
        compiler_params=pltpu.CompilerParams(use_tc_tiling_on_sc=False),
        out_type=jax.ShapeDtypeStruct((m, d), jnp.float32),
        scratch_types=[
            pltpu.VMEM((bpw,), jnp.int32),
            pltpu.VMEM((bpw, d), jnp.float32),
            pltpu.SemaphoreType.DMA,
        ],
    )
    def gather(cb_hbm, idx_hbm, out_hbm, idx_v, rows_v, sem):
        wid = lax.axis_index("s") * 2 + lax.axis_index("c")
        base = wid * bpw
        pltpu.sync_copy(idx_hbm.at[pl.ds(base, bpw)], idx_v)
        copies = [
            pltpu.async_copy(
                cb_hbm.at[idx_v.at[pl.ds(j * chunk, chunk)]],
                rows_v.at[pl.ds(j * chunk, chunk), :],
                sem,
            )
            for j in range(nch)
        ]
        for c in copies:
            c.wait()
        pltpu.sync_copy(rows_v, out_hbm.at[pl.ds(base, bpw), :])

    return gather(codebook, idx_flat)


def kernel(z, codebook):
    shape = z.shape
    d = shape[-1]
    m = z.size // d
    zf = z.reshape(m, d)
    q = _argmax_tc(zf, codebook)
    emb = _gather_sc(codebook, q)
    return emb.reshape(shape), q.reshape(shape[:-1] + (1,))


# speedup vs baseline: 1.2822x; 1.2822x over previous
import functools

import jax
import jax.numpy as jnp
from jax import lax
from jax.experimental import pallas as pl
from jax.experimental.pallas import tpu as pltpu
from jax.experimental.pallas import tpu_sc as plsc

_BM = 2048


def _vq_body(z_ref, cb_ref, q_ref):
    zb = z_ref[...]
    cb = cb_ref[...]
    zn = zb / jnp.maximum(
        jnp.sqrt(jnp.sum(zb * zb, axis=-1, keepdims=True)), 1e-12)
    en = cb / jnp.maximum(
        jnp.sqrt(jnp.sum(cb * cb, axis=-1, keepdims=True)), 1e-12)
    sims = jax.lax.dot_general(
        en, zn, (((1,), (1,)), ((), ())),
        precision=jax.lax.Precision.DEFAULT)
    k = sims.shape[0]
    row = jax.lax.broadcasted_iota(jnp.int32, sims.shape, 0)
    sims = jnp.where(row == 0, -jnp.inf, sims)
    m = jnp.max(sims, axis=0, keepdims=True)
    qi = jnp.min(jnp.where(sims == m, row, k), axis=0)
    q_ref[...] = qi


def _argmax_tc(zf, codebook):
    m, d = zf.shape
    k = codebook.shape[0]
    return pl.pallas_call(
        _vq_body,
        grid=(m // _BM,),
        in_specs=[
            pl.BlockSpec((_BM, d), lambda i: (i, 0)),
            pl.BlockSpec((k, d), lambda i: (0, 0)),
        ],
        out_specs=pl.BlockSpec((_BM,), lambda i: (i,)),
        out_shape=jax.ShapeDtypeStruct((m,), jnp.int32),
    )(zf, codebook)


def _gather_sc(codebook, idx_flat):
    m = idx_flat.shape[0]
    d = codebook.shape[1]
    nw = 2 * 16
    bpw = m // nw
    chunk = 96
    nch = bpw // chunk
    mesh = plsc.VectorSubcoreMesh(core_axis_name="c", subcore_axis_name="s")

    @functools.partial(
        pl.kernel, mesh=mesh,
        compiler_params=pltpu.CompilerParams(use_tc_tiling_on_sc=False),
        out_type=jax.ShapeDtypeStruct((m, d), jnp.float32),
        scratch_types=[
            pltpu.VMEM((bpw,), jnp.int32),
            pltpu.VMEM((bpw, d), jnp.float32),
            pltpu.SemaphoreType.DMA,
        ],
    )
    def gather(cb_hbm, idx_hbm, out_hbm, idx_v, rows_v, sem):
        wid = lax.axis_index("s") * 2 + lax.axis_index("c")
        base = wid * bpw
        pltpu.sync_copy(idx_hbm.at[pl.ds(base, bpw)], idx_v)
        copies = [
            pltpu.async_copy(
                cb_hbm.at[idx_v.at[pl.ds(j * chunk, chunk)]],
                rows_v.at[pl.ds(j * chunk, chunk), :],
                sem,
            )
            for j in range(nch)
        ]
        for c in copies:
            c.wait()
        pltpu.sync_copy(rows_v, out_hbm.at[pl.ds(base, bpw), :])

    return gather(codebook, idx_flat)


def kernel(z, codebook):
    shape = z.shape
    d = shape[-1]
    m = z.size // d
    zf = z.reshape(m, d)
    q = _argmax_tc(zf, codebook)
    emb = _gather_sc(codebook, q)
    return emb.reshape(shape), q.reshape(shape[:-1] + (1,))


# speedup vs baseline: 1.3083x; 1.0204x over previous
import functools

import jax
import jax.numpy as jnp
from jax import lax
from jax.experimental import pallas as pl
from jax.experimental.pallas import tpu as pltpu
from jax.experimental.pallas import tpu_sc as plsc

_B0 = 8


def _vq_body(z_ref, cb_ref, q_ref):
    zb3 = z_ref[...]
    zb = zb3.reshape(zb3.shape[0] * zb3.shape[1], zb3.shape[2])
    cb = cb_ref[...]
    zn = zb / jnp.maximum(
        jnp.sqrt(jnp.sum(zb * zb, axis=-1, keepdims=True)), 1e-12)
    en = cb / jnp.maximum(
        jnp.sqrt(jnp.sum(cb * cb, axis=-1, keepdims=True)), 1e-12)
    sims = jax.lax.dot_general(
        en, zn, (((1,), (1,)), ((), ())),
        precision=jax.lax.Precision.DEFAULT)
    k = sims.shape[0]
    row = jax.lax.broadcasted_iota(jnp.int32, sims.shape, 0)
    sims = jnp.where(row == 0, -jnp.inf, sims)
    m = jnp.max(sims, axis=0, keepdims=True)
    qi = jnp.min(jnp.where(sims == m, row, k), axis=0)
    rows = qi.shape[0]
    q_ref[pl.ds(pl.program_id(0) * rows, rows)] = qi


def _argmax_tc(z, codebook):
    b, n, d = z.shape
    k = codebook.shape[0]
    m = b * n
    return pl.pallas_call(
        _vq_body,
        grid=(b // _B0,),
        in_specs=[
            pl.BlockSpec((_B0, n, d), lambda i: (i, 0, 0)),
            pl.BlockSpec((k, d), lambda i: (0, 0)),
        ],
        out_specs=pl.BlockSpec((m,), lambda i: (0,)),
        out_shape=jax.ShapeDtypeStruct((m,), jnp.int32),
    )(z, codebook)


def _gather_sc(codebook, idx_flat):
    m = idx_flat.shape[0]
    d = codebook.shape[1]
    nw = 2 * 16
    bpw = m // nw
    chunk = 96
    nch = bpw // chunk
    mesh = plsc.VectorSubcoreMesh(core_axis_name="c", subcore_axis_name="s")

    @functools.partial(
        pl.kernel, mesh=mesh,
        compiler_params=pltpu.CompilerParams(use_tc_tiling_on_sc=False),
        out_type=jax.ShapeDtypeStruct((m, d), jnp.float32),
        scratch_types=[
            pltpu.VMEM((bpw,), jnp.int32),
            pltpu.VMEM((bpw, d), jnp.float32),
            pltpu.SemaphoreType.DMA,
        ],
    )
    def gather(cb_hbm, idx_hbm, out_hbm, idx_v, rows_v, sem):
        wid = lax.axis_index("s") * 2 + lax.axis_index("c")
        base = wid * bpw
        pltpu.sync_copy(idx_hbm.at[pl.ds(base, bpw)], idx_v)
        copies = [
            pltpu.async_copy(
                cb_hbm.at[idx_v.at[pl.ds(j * chunk, chunk)]],
                rows_v.at[pl.ds(j * chunk, chunk), :],
                sem,
            )
            for j in range(nch)
        ]
        for c in copies:
            c.wait()
        pltpu.sync_copy(rows_v, out_hbm.at[pl.ds(base, bpw), :])

    return gather(codebook, idx_flat)


def kernel(z, codebook):
    shape = z.shape
    d = shape[-1]
    m = z.size // d
    q = _argmax_tc(z, codebook)
    emb = _gather_sc(codebook, q)
    return emb.reshape(shape), q.reshape(shape[:-1] + (1,))


# speedup vs baseline: 1.3885x; 1.0613x over previous
import functools

import jax
import jax.numpy as jnp
from jax import lax
from jax.experimental import pallas as pl
from jax.experimental.pallas import tpu as pltpu
from jax.experimental.pallas import tpu_sc as plsc

_B0 = 2


def _vq_body(zt_ref, cbt_ref, q_ref):
    cbt = cbt_ref[...]
    en = cbt / jnp.maximum(
        jnp.sqrt(jnp.sum(cbt * cbt, axis=0, keepdims=True)), 1e-12)
    k = cbt.shape[1]
    qs = []
    for b in range(_B0):
        zb = zt_ref[b]
        zn = zb / jnp.maximum(
            jnp.sqrt(jnp.sum(zb * zb, axis=0, keepdims=True)), 1e-12)
        sims = jax.lax.dot_general(
            en, zn, (((0,), (0,)), ((), ())),
            precision=jax.lax.Precision.DEFAULT)
        row = jax.lax.broadcasted_iota(jnp.int32, sims.shape, 0)
        sims = jnp.where(row == 0, -jnp.inf, sims)
        m = jnp.max(sims, axis=0, keepdims=True)
        qs.append(jnp.min(jnp.where(sims == m, row, k), axis=0))
    qi = jnp.concatenate(qs)
    rows = qi.shape[0]
    q_ref[pl.ds(pl.program_id(0) * rows, rows)] = qi


def _argmax_tc(zt, cbt):
    b, d, n = zt.shape
    k = cbt.shape[1]
    m = b * n
    return pl.pallas_call(
        _vq_body,
        grid=(b // _B0,),
        in_specs=[
            pl.BlockSpec((_B0, d, n), lambda i: (i, 0, 0)),
            pl.BlockSpec((d, k), lambda i: (0, 0)),
        ],
        out_specs=pl.BlockSpec((m,), lambda i: (0,)),
        out_shape=jax.ShapeDtypeStruct((m,), jnp.int32),
    )(zt, cbt)


def _gather_sc(codebook, idx_flat, b, n):
    d = codebook.shape[1]
    bpw = n
    chunk = 96
    nch = bpw // chunk
    mesh = plsc.VectorSubcoreMesh(core_axis_name="c", subcore_axis_name="s")

    @functools.partial(
        pl.kernel, mesh=mesh,
        compiler_params=pltpu.CompilerParams(use_tc_tiling_on_sc=False),
        out_type=jax.ShapeDtypeStruct((b, n, d), jnp.float32),
        scratch_types=[
            pltpu.VMEM((bpw,), jnp.int32),
            pltpu.VMEM((bpw, d), jnp.float32),
            pltpu.SemaphoreType.DMA,
        ],
    )
    def gather(cb_hbm, idx_hbm, out_hbm, idx_v, rows_v, sem):
        wid = lax.axis_index("s") * 2 + lax.axis_index("c")
        pltpu.sync_copy(idx_hbm.at[pl.ds(wid * bpw, bpw)], idx_v)
        copies = [
            pltpu.async_copy(
                cb_hbm.at[idx_v.at[pl.ds(j * chunk, chunk)]],
                rows_v.at[pl.ds(j * chunk, chunk), :],
                sem,
            )
            for j in range(nch)
        ]
        for c in copies:
            c.wait()
        pltpu.sync_copy(rows_v, out_hbm.at[wid])

    return gather(codebook, idx_flat)


def kernel(z, codebook):
    b, n, d = z.shape
    zt = jnp.swapaxes(z, 1, 2)
    cbt = codebook.T
    q = _argmax_tc(zt, cbt)
    emb = _gather_sc(codebook, q, b, n)
    return emb, q.reshape(b, n, 1)


# speedup vs baseline: 1.4240x; 1.0256x over previous
import functools

import jax
import jax.numpy as jnp
from jax import lax
from jax.experimental import pallas as pl
from jax.experimental.pallas import tpu as pltpu
from jax.experimental.pallas import tpu_sc as plsc

_B0 = 4


def _vq_body(zt_ref, cbt_ref, q_ref):
    cbt = cbt_ref[...]
    en = cbt / jnp.maximum(
        jnp.sqrt(jnp.sum(cbt * cbt, axis=0, keepdims=True)), 1e-12)
    k = cbt.shape[1]
    qs = []
    for b in range(_B0):
        zb = zt_ref[b]
        zn = zb / jnp.maximum(
            jnp.sqrt(jnp.sum(zb * zb, axis=0, keepdims=True)), 1e-12)
        sims = jax.lax.dot_general(
            en, zn, (((0,), (0,)), ((), ())),
            precision=jax.lax.Precision.DEFAULT)
        row = jax.lax.broadcasted_iota(jnp.int32, sims.shape, 0)
        sims = jnp.where(row == 0, -jnp.inf, sims)
        m = jnp.max(sims, axis=0, keepdims=True)
        qs.append(jnp.min(jnp.where(sims == m, row, k), axis=0))
    qi = jnp.concatenate(qs)
    rows = qi.shape[0]
    q_ref[pl.ds(pl.program_id(0) * rows, rows)] = qi


def _argmax_tc(zt, cbt):
    b, d, n = zt.shape
    k = cbt.shape[1]
    m = b * n
    return pl.pallas_call(
        _vq_body,
        grid=(b // _B0,),
        in_specs=[
            pl.BlockSpec((_B0, d, n), lambda i: (i, 0, 0)),
            pl.BlockSpec((d, k), lambda i: (0, 0)),
        ],
        out_specs=pl.BlockSpec((m,), lambda i: (0,)),
        out_shape=jax.ShapeDtypeStruct((m,), jnp.int32),
    )(zt, cbt)


def _gather_sc(codebook, idx_flat, b, n):
    d = codebook.shape[1]
    bpw = n
    chunk = 96
    nch = bpw // chunk
    mesh = plsc.VectorSubcoreMesh(core_axis_name="c", subcore_axis_name="s")

    @functools.partial(
        pl.kernel, mesh=mesh,
        compiler_params=pltpu.CompilerParams(use_tc_tiling_on_sc=False),
        out_type=jax.ShapeDtypeStruct((b, n, d), jnp.float32),
        scratch_types=[
            pltpu.VMEM((bpw,), jnp.int32),
            pltpu.VMEM((bpw, d), jnp.float32),
            pltpu.SemaphoreType.DMA,
        ],
    )
    def gather(cb_hbm, idx_hbm, out_hbm, idx_v, rows_v, sem):
        wid = lax.axis_index("s") * 2 + lax.axis_index("c")
        pltpu.sync_copy(idx_hbm.at[pl.ds(wid * bpw, bpw)], idx_v)
        copies = [
            pltpu.async_copy(
                cb_hbm.at[idx_v.at[pl.ds(j * chunk, chunk)]],
                rows_v.at[pl.ds(j * chunk, chunk), :],
                sem,
            )
            for j in range(nch)
        ]
        for c in copies:
            c.wait()
        pltpu.sync_copy(rows_v, out_hbm.at[wid])

    return gather(codebook, idx_flat)


def kernel(z, codebook):
    b, n, d = z.shape
    zt = jnp.swapaxes(z, 1, 2)
    cbt = codebook.T
    q = _argmax_tc(zt, cbt)
    emb = _gather_sc(codebook, q, b, n)
    return emb, q.reshape(b, n, 1)


# speedup vs baseline: 1.4509x; 1.0189x over previous
import functools

import jax
import jax.numpy as jnp
from jax import lax
from jax.experimental import pallas as pl
from jax.experimental.pallas import tpu as pltpu
from jax.experimental.pallas import tpu_sc as plsc

_B0 = 8


def _vq_body(zt_ref, cbt_ref, q_ref):
    cbt = cbt_ref[...]
    en = cbt / jnp.maximum(
        jnp.sqrt(jnp.sum(cbt * cbt, axis=0, keepdims=True)), 1e-12)
    k = cbt.shape[1]
    qs = []
    for b in range(_B0):
        zb = zt_ref[b]
        zn = zb / jnp.maximum(
            jnp.sqrt(jnp.sum(zb * zb, axis=0, keepdims=True)), 1e-12)
        sims = jax.lax.dot_general(
            en, zn, (((0,), (0,)), ((), ())),
            precision=jax.lax.Precision.DEFAULT)
        row = jax.lax.broadcasted_iota(jnp.int32, sims.shape, 0)
        sims = jnp.where(row == 0, -jnp.inf, sims)
        m = jnp.max(sims, axis=0, keepdims=True)
        qs.append(jnp.min(jnp.where(sims == m, row, k), axis=0))
    qi = jnp.concatenate(qs)
    rows = qi.shape[0]
    q_ref[pl.ds(pl.program_id(0) * rows, rows)] = qi


def _argmax_tc(zt, cbt):
    b, d, n = zt.shape
    k = cbt.shape[1]
    m = b * n
    return pl.pallas_call(
        _vq_body,
        grid=(b // _B0,),
        in_specs=[
            pl.BlockSpec((_B0, d, n), lambda i: (i, 0, 0)),
            pl.BlockSpec((d, k), lambda i: (0, 0)),
        ],
        out_specs=pl.BlockSpec((m,), lambda i: (0,)),
        out_shape=jax.ShapeDtypeStruct((m,), jnp.int32),
    )(zt, cbt)


def _gather_sc(codebook, idx_flat, b, n):
    d = codebook.shape[1]
    bpw = n
    chunk = 96
    nch = bpw // chunk
    mesh = plsc.VectorSubcoreMesh(core_axis_name="c", subcore_axis_name="s")

    @functools.partial(
        pl.kernel, mesh=mesh,
        compiler_params=pltpu.CompilerParams(use_tc_tiling_on_sc=False),
        out_type=jax.ShapeDtypeStruct((b, n, d), jnp.float32),
        scratch_types=[
            pltpu.VMEM((bpw,), jnp.int32),
            pltpu.VMEM((bpw, d), jnp.float32),
            pltpu.SemaphoreType.DMA,
        ],
    )
    def gather(cb_hbm, idx_hbm, out_hbm, idx_v, rows_v, sem):
        wid = lax.axis_index("s") * 2 + lax.axis_index("c")
        pltpu.sync_copy(idx_hbm.at[pl.ds(wid * bpw, bpw)], idx_v)
        copies = [
            pltpu.async_copy(
                cb_hbm.at[idx_v.at[pl.ds(j * chunk, chunk)]],
                rows_v.at[pl.ds(j * chunk, chunk), :],
                sem,
            )
            for j in range(nch)
        ]
        for c in copies:
            c.wait()
        pltpu.sync_copy(rows_v, out_hbm.at[wid])

    return gather(codebook, idx_flat)


def kernel(z, codebook):
    b, n, d = z.shape
    zt = jnp.swapaxes(z, 1, 2)
    cbt = codebook.T
    q = _argmax_tc(zt, cbt)
    emb = _gather_sc(codebook, q, b, n)
    return emb, q.reshape(b, n, 1)
